# baseline (speedup 1.0000x reference)
import functools

import jax
import jax.numpy as jnp
from jax import lax
from jax.experimental import pallas as pl
from jax.experimental.pallas import tpu as pltpu
from jax.experimental.pallas import tpu_sc as plsc

N = 10000
E = 160000
D = 256
GENE = 512
H = 256
H2 = 128
NTRAIN = 20000

NSC = 2
NT = 16
CH = 128
TROWS = 632
NP = NT * TROWS
EC = 80
EPAD = NT * EC * CH
DC = 40
DPAD = NT * DC * CH
PC = 5
PPAD = NSC * NT * PC * CH

_mesh = plsc.VectorSubcoreMesh(core_axis_name="c", subcore_axis_name="s")


def _deg_body(idx_hbm, ones_hbm, out_hbm, idx_v, ones_v, accum):
  c = lax.axis_index("c")
  t = lax.axis_index("s")
  pltpu.sync_copy(idx_hbm.at[c, t], idx_v)
  pltpu.sync_copy(ones_hbm.at[pl.ds(0, CH)], ones_v)
  pltpu.sync_copy(ones_hbm, accum.at[pl.ds(t * TROWS, TROWS)])
  plsc.subcore_barrier()

  def body(j, carry):
    pltpu.sync_copy(ones_v, accum.at[idx_v.at[j]], add=True)
    return carry

  lax.fori_loop(0, DC, body, 0)
  plsc.subcore_barrier()
  pltpu.sync_copy(accum.at[pl.ds(t * TROWS, TROWS)],
                  out_hbm.at[c, pl.ds(t * TROWS, TROWS)])


_deg_kernel = pl.kernel(
    _deg_body,
    out_type=jax.ShapeDtypeStruct((NSC, NP, CH), jnp.float32),
    mesh=_mesh,
    scratch_types=[
        pltpu.VMEM((DC, CH), jnp.int32),
        pltpu.VMEM((CH, CH), jnp.float32),
        pltpu.VMEM_SHARED((NP, CH), jnp.float32),
    ],
)


G = 8
NG = EC // G


def _edge_body(xs_hbm, src_hbm, dst_hbm, out_hbm, sring, dring, buf0, buf1,
               accum, sem0, sem1):
  c = lax.axis_index("c")
  t = lax.axis_index("s")
  pltpu.sync_copy(xs_hbm.at[pl.ds(c * NP + t * TROWS, TROWS)],
                  accum.at[pl.ds(t * TROWS, TROWS)])
  plsc.subcore_barrier()

  def group(g, carry):
    pltpu.sync_copy(src_hbm.at[c, t, pl.ds(g * G, G)], sring)
    pltpu.sync_copy(dst_hbm.at[t, pl.ds(g * G, G)], dring)
    for k in range(G):
      pltpu.async_copy(xs_hbm.at[sring.at[k]], buf0, sem0).wait()
      pltpu.sync_copy(buf0, accum.at[dring.at[k]], add=True)
    return carry

  lax.fori_loop(0, NG, group, 0)
  plsc.subcore_barrier()
  pltpu.sync_copy(accum.at[pl.ds(t * TROWS, TROWS)],
                  out_hbm.at[c, pl.ds(t * TROWS, TROWS)])


_edge_kernel = pl.kernel(
    _edge_body,
    out_type=jax.ShapeDtypeStruct((NSC, NP, H // NSC), jnp.float32),
    mesh=_mesh,
    scratch_types=[
        pltpu.VMEM((G, CH), jnp.int32),
        pltpu.VMEM((G, CH), jnp.int32),
        pltpu.VMEM((CH, H // NSC), jnp.float32),
        pltpu.VMEM((CH, H // NSC), jnp.float32),
        pltpu.VMEM_SHARED((NP, H // NSC), jnp.float32),
        pltpu.SemaphoreType.DMA,
        pltpu.SemaphoreType.DMA,
    ],
)


def _pair_body(z_hbm, fidx_hbm, gidx_hbm, outf_hbm, outg_hbm, fidx_v, gidx_v,
               buf0, sem):
  c = lax.axis_index("c")
  t = lax.axis_index("s")
  w = c * NT + t
  pltpu.sync_copy(fidx_hbm.at[c, t], fidx_v)
  pltpu.sync_copy(gidx_hbm.at[c, t], gidx_v)

  def body(j, carry):
    base = w * (PC * CH) + j * CH
    pltpu.async_copy(z_hbm.at[fidx_v.at[j]], buf0, sem).wait()
    pltpu.sync_copy(buf0, outf_hbm.at[pl.ds(base, CH)])
    pltpu.async_copy(z_hbm.at[gidx_v.at[j]], buf0, sem).wait()
    pltpu.sync_copy(buf0, outg_hbm.at[pl.ds(base, CH)])
    return carry

  lax.fori_loop(0, PC, body, 0)


_pair_kernel = pl.kernel(
    _pair_body,
    out_type=[
        jax.ShapeDtypeStruct((PPAD, H2), jnp.float32),
        jax.ShapeDtypeStruct((PPAD, H2), jnp.float32),
    ],
    mesh=_mesh,
    scratch_types=[
        pltpu.VMEM((PC, CH), jnp.int32),
        pltpu.VMEM((PC, CH), jnp.int32),
        pltpu.VMEM((CH, H2), jnp.float32),
        pltpu.SemaphoreType.DMA,
    ],
)


BM = 400


def _mm1_body(x_ref, w_ref, degp_ref, out_ref, dinv_ref):
  deg = degp_ref[0, :, 0:1] + degp_ref[1, :, 0:1] - 1.0
  dinv = lax.rsqrt(deg)
  dinv_ref[...] = dinv
  xw = jnp.dot(x_ref[...], w_ref[...], preferred_element_type=jnp.float32)
  xs = xw * dinv
  out_ref[0] = xs[:, :H // 2]
  out_ref[1] = xs[:, H // 2:]


def _mm2_body(acc_ref, w_ref, b_ref, dinv_ref, out_ref):
  dinv = dinv_ref[...]
  h = jnp.concatenate([acc_ref[0], acc_ref[1]], axis=1) * dinv + b_ref[...]
  h = jnp.maximum(h, 0.0)
  xw = jnp.dot(h, w_ref[...], preferred_element_type=jnp.float32)
  xs = xw * dinv
  out_ref[0] = xs[:, :H // 2]
  out_ref[1] = xs[:, H // 2:]


def _lrelu(x):
  return jnp.where(x > 0, x, 0.01 * x)


def _mlp_body(acc_ref, dinv_ref, b2_ref, llm_ref, m1l_ref, m1r_ref, c1_ref,
              m2_ref, c2_ref, out_ref):
  dinv = dinv_ref[...]
  h = jnp.concatenate([acc_ref[0], acc_ref[1]], axis=1) * dinv + b2_ref[...]
  e = (jnp.dot(llm_ref[...], m1l_ref[...], preferred_element_type=jnp.float32)
       + jnp.dot(h, m1r_ref[...], preferred_element_type=jnp.float32)
       + c1_ref[...])
  z1 = _lrelu(e)
  z2 = _lrelu(jnp.dot(z1, m2_ref[...], preferred_element_type=jnp.float32)
              + c2_ref[...])
  out_ref[...] = z2


def _dot_body(tf_ref, tg_ref, out_ref):
  out_ref[...] = jnp.sum(tf_ref[...] * tg_ref[...], axis=1, keepdims=True)


def _row_spec(shape):
  nd = len(shape)
  return pl.BlockSpec((None,) * 0 + shape, lambda i: (i,) + (0,) * (nd - 1))


def _full_spec(shape):
  nd = len(shape)
  return pl.BlockSpec(shape, lambda i: (0,) * nd)


_mm1 = pl.pallas_call(
    _mm1_body,
    grid=(N // BM,),
    in_specs=[
        _row_spec((BM, D)),
        _full_spec((D, H)),
        pl.BlockSpec((2, BM, CH), lambda i: (0, i, 0)),
    ],
    out_specs=[
        pl.BlockSpec((2, BM, H // 2), lambda i: (0, i, 0)),
        pl.BlockSpec((BM, 1), lambda i: (i, 0)),
    ],
    out_shape=[
        jax.ShapeDtypeStruct((2, NP, H // 2), jnp.float32),
        jax.ShapeDtypeStruct((NP, 1), jnp.float32),
    ],
)

_mm2 = pl.pallas_call(
    _mm2_body,
    grid=(N // BM,),
    in_specs=[
        pl.BlockSpec((2, BM, H // 2), lambda i: (0, i, 0)),
        _full_spec((H, H)),
        _full_spec((1, H)),
        pl.BlockSpec((BM, 1), lambda i: (i, 0)),
    ],
    out_specs=pl.BlockSpec((2, BM, H // 2), lambda i: (0, i, 0)),
    out_shape=jax.ShapeDtypeStruct((2, NP, H // 2), jnp.float32),
)

_mlp = pl.pallas_call(
    _mlp_body,
    grid=(N // BM,),
    in_specs=[
        pl.BlockSpec((2, BM, H // 2), lambda i: (0, i, 0)),
        pl.BlockSpec((BM, 1), lambda i: (i, 0)),
        _full_spec((1, H)),
        _row_spec((BM, GENE)),
        _full_spec((GENE, H)),
        _full_spec((H, H)),
        _full_spec((1, H)),
        _full_spec((H, H2)),
        _full_spec((1, H2)),
    ],
    out_specs=_row_spec((BM, H2)),
    out_shape=jax.ShapeDtypeStruct((N, H2), jnp.float32),
)

_BP = 512

_pair_dot = pl.pallas_call(
    _dot_body,
    grid=(PPAD // _BP,),
    in_specs=[_row_spec((_BP, H2)), _row_spec((_BP, H2))],
    out_specs=pl.BlockSpec((_BP, 1), lambda i: (i, 0)),
    out_shape=jax.ShapeDtypeStruct((PPAD, 1), jnp.float32),
)


def _pad_reshape(a, total, fill, shape):
  pad = jnp.full((total - a.shape[0],), fill, dtype=a.dtype)
  return jnp.concatenate([a, pad]).reshape(shape)


def _pad_spread(a, total, shape):
  n = total - a.shape[0]
  pad = N + (jnp.arange(n, dtype=a.dtype) % (NP - N))
  return jnp.concatenate([a, pad]).reshape(shape)


@jax.jit
def kernel(x, adj, train_sample, llm_emb, W1, b1, W2, b2, M1, c1, M2, c2):
  src = adj[0]
  dst = adj[1]

  src_pad = _pad_reshape(src, EPAD, 0, (NT, EC, CH))
  src_idx = jnp.stack([src_pad, src_pad + NP])
  dst_idx = _pad_spread(dst, EPAD, (NT, EC, CH))
  deg_idx = jnp.stack([
      _pad_spread(dst[:E // 2], DPAD, (NT, DC, CH)),
      _pad_spread(dst[E // 2:], DPAD, (NT, DC, CH)),
  ])
  fidx = _pad_reshape(train_sample[:, 0], PPAD, 0, (NSC, NT, PC, CH))
  gidx = _pad_reshape(train_sample[:, 1], PPAD, 0, (NSC, NT, PC, CH))
  ones = jnp.ones((TROWS, CH), jnp.float32)

  degp = _deg_kernel(deg_idx, ones)

  xs1, dinv = _mm1(x, W1, degp)
  xs1 = xs1.reshape(NSC * NP, H // 2)
  acc1 = _edge_kernel(xs1, src_idx, dst_idx)

  xs2 = _mm2(acc1, W2, b1.reshape(1, H), dinv).reshape(NSC * NP, H // 2)
  acc2 = _edge_kernel(xs2, src_idx, dst_idx)

  z = _mlp(acc2, dinv, b2.reshape(1, H), llm_emb,
           M1[:, :GENE].T, M1[:, GENE:].T, c1.reshape(1, H),
           M2.T, c2.reshape(1, H2))

  tf, tg = _pair_kernel(z, fidx, gidx)
  pred = _pair_dot(tf, tg)
  return pred[:NTRAIN]


# speedup vs baseline: 1.5872x; 1.5872x over previous
import functools

import jax
import jax.numpy as jnp
from jax import lax
from jax.experimental import pallas as pl
from jax.experimental.pallas import tpu as pltpu
from jax.experimental.pallas import tpu_sc as plsc

N = 10000
E = 160000
D = 256
GENE = 512
H = 256
H2 = 128
NTRAIN = 20000

NSC = 2
NT = 16
CH = 128
TROWS = 632
NP = NT * TROWS
EC = 79
EPAD = NT * EC * CH
DC = 40
DPAD = NT * DC * CH
PC = 5
PPAD = NSC * NT * PC * CH

_mesh = plsc.VectorSubcoreMesh(core_axis_name="c", subcore_axis_name="s")


def _deg_body(idx_hbm, ones_hbm, out_hbm, idx_v, ones_v, accum):
  c = lax.axis_index("c")
  t = lax.axis_index("s")
  pltpu.sync_copy(idx_hbm.at[c, t], idx_v)
  pltpu.sync_copy(ones_hbm.at[pl.ds(0, CH)], ones_v)
  pltpu.sync_copy(ones_hbm, accum.at[pl.ds(t * TROWS, TROWS)])
  plsc.subcore_barrier()

  def body(j, carry):
    pltpu.sync_copy(ones_v, accum.at[idx_v.at[j]], add=True)
    return carry

  lax.fori_loop(0, DC, body, 0)
  plsc.subcore_barrier()
  pltpu.sync_copy(accum.at[pl.ds(t * TROWS, TROWS)],
                  out_hbm.at[c, pl.ds(t * TROWS, TROWS)])


_deg_kernel = pl.kernel(
    _deg_body,
    out_type=jax.ShapeDtypeStruct((NSC, NP, CH), jnp.float32),
    mesh=_mesh,
    scratch_types=[
        pltpu.VMEM((DC, CH), jnp.int32),
        pltpu.VMEM((CH, CH), jnp.float32),
        pltpu.VMEM_SHARED((NP, CH), jnp.float32),
    ],
)


G = 79
NG = EC // G


def _edge_body(xs_hbm, src_hbm, dst_hbm, out_hbm, sring, dring, buf0, buf1,
               accum, sem0, sem1):
  c = lax.axis_index("c")
  t = lax.axis_index("s")
  pltpu.sync_copy(xs_hbm.at[pl.ds(c * NP + t * TROWS, TROWS)],
                  accum.at[pl.ds(t * TROWS, TROWS)])
  plsc.subcore_barrier()

  def group(g, carry):
    pltpu.sync_copy(src_hbm.at[c, t, pl.ds(g * G, G)], sring)
    pltpu.sync_copy(dst_hbm.at[t, pl.ds(g * G, G)], dring)
    for k in range(G):
      pltpu.async_copy(xs_hbm.at[sring.at[k]], buf0, sem0).wait()
      pltpu.sync_copy(buf0, accum.at[dring.at[k]], add=True)
    return carry

  lax.fori_loop(0, NG, group, 0)
  plsc.subcore_barrier()
  pltpu.sync_copy(accum.at[pl.ds(t * TROWS, TROWS)],
                  out_hbm.at[c, pl.ds(t * TROWS, TROWS)])


_edge_kernel = pl.kernel(
    _edge_body,
    out_type=jax.ShapeDtypeStruct((NSC, NP, H // NSC), jnp.float32),
    mesh=_mesh,
    scratch_types=[
        pltpu.VMEM((G, CH), jnp.int32),
        pltpu.VMEM((G, CH), jnp.int32),
        pltpu.VMEM((CH, H // NSC), jnp.float32),
        pltpu.VMEM((CH, H // NSC), jnp.float32),
        pltpu.VMEM_SHARED((NP, H // NSC), jnp.float32),
        pltpu.SemaphoreType.DMA,
        pltpu.SemaphoreType.DMA,
    ],
)


def _pair_body(z_hbm, fidx_hbm, gidx_hbm, outf_hbm, outg_hbm, fidx_v, gidx_v,
               buf0, sem):
  c = lax.axis_index("c")
  t = lax.axis_index("s")
  w = c * NT + t
  pltpu.sync_copy(fidx_hbm.at[c, t], fidx_v)
  pltpu.sync_copy(gidx_hbm.at[c, t], gidx_v)

  def body(j, carry):
    base = w * (PC * CH) + j * CH
    pltpu.async_copy(z_hbm.at[fidx_v.at[j]], buf0, sem).wait()
    pltpu.sync_copy(buf0, outf_hbm.at[pl.ds(base, CH)])
    pltpu.async_copy(z_hbm.at[gidx_v.at[j]], buf0, sem).wait()
    pltpu.sync_copy(buf0, outg_hbm.at[pl.ds(base, CH)])
    return carry

  lax.fori_loop(0, PC, body, 0)


_pair_kernel = pl.kernel(
    _pair_body,
    out_type=[
        jax.ShapeDtypeStruct((PPAD, H2), jnp.float32),
        jax.ShapeDtypeStruct((PPAD, H2), jnp.float32),
    ],
    mesh=_mesh,
    scratch_types=[
        pltpu.VMEM((PC, CH), jnp.int32),
        pltpu.VMEM((PC, CH), jnp.int32),
        pltpu.VMEM((CH, H2), jnp.float32),
        pltpu.SemaphoreType.DMA,
    ],
)


BM = 400


def _mm1_body(x_ref, w_ref, degp_ref, out_ref, dinv_ref):
  deg = degp_ref[0, :, 0:1] + degp_ref[1, :, 0:1] - 1.0
  dinv = lax.rsqrt(deg)
  dinv_ref[...] = dinv
  xw = jnp.dot(x_ref[...], w_ref[...], preferred_element_type=jnp.float32)
  xs = xw * dinv
  out_ref[0] = xs[:, :H // 2]
  out_ref[1] = xs[:, H // 2:]


def _mm2_body(acc_ref, w_ref, b_ref, dinv_ref, out_ref):
  dinv = dinv_ref[...]
  h = jnp.concatenate([acc_ref[0], acc_ref[1]], axis=1) * dinv + b_ref[...]
  h = jnp.maximum(h, 0.0)
  xw = jnp.dot(h, w_ref[...], preferred_element_type=jnp.float32)
  xs = xw * dinv
  out_ref[0] = xs[:, :H // 2]
  out_ref[1] = xs[:, H // 2:]


def _lrelu(x):
  return jnp.where(x > 0, x, 0.01 * x)


def _mlp_body(acc_ref, dinv_ref, b2_ref, llm_ref, m1l_ref, m1r_ref, c1_ref,
              m2_ref, c2_ref, out_ref):
  dinv = dinv_ref[...]
  h = jnp.concatenate([acc_ref[0], acc_ref[1]], axis=1) * dinv + b2_ref[...]
  e = (jnp.dot(llm_ref[...], m1l_ref[...], preferred_element_type=jnp.float32)
       + jnp.dot(h, m1r_ref[...], preferred_element_type=jnp.float32)
       + c1_ref[...])
  z1 = _lrelu(e)
  z2 = _lrelu(jnp.dot(z1, m2_ref[...], preferred_element_type=jnp.float32)
              + c2_ref[...])
  out_ref[...] = z2


def _dot_body(tf_ref, tg_ref, out_ref):
  out_ref[...] = jnp.sum(tf_ref[...] * tg_ref[...], axis=1, keepdims=True)


def _row_spec(shape):
  nd = len(shape)
  return pl.BlockSpec((None,) * 0 + shape, lambda i: (i,) + (0,) * (nd - 1))


def _full_spec(shape):
  nd = len(shape)
  return pl.BlockSpec(shape, lambda i: (0,) * nd)


_mm1 = pl.pallas_call(
    _mm1_body,
    grid=(N // BM,),
    in_specs=[
        _row_spec((BM, D)),
        _full_spec((D, H)),
        pl.BlockSpec((2, BM, CH), lambda i: (0, i, 0)),
    ],
    out_specs=[
        pl.BlockSpec((2, BM, H // 2), lambda i: (0, i, 0)),
        pl.BlockSpec((BM, 1), lambda i: (i, 0)),
    ],
    out_shape=[
        jax.ShapeDtypeStruct((2, NP, H // 2), jnp.float32),
        jax.ShapeDtypeStruct((NP, 1), jnp.float32),
    ],
)

_mm2 = pl.pallas_call(
    _mm2_body,
    grid=(N // BM,),
    in_specs=[
        pl.BlockSpec((2, BM, H // 2), lambda i: (0, i, 0)),
        _full_spec((H, H)),
        _full_spec((1, H)),
        pl.BlockSpec((BM, 1), lambda i: (i, 0)),
    ],
    out_specs=pl.BlockSpec((2, BM, H // 2), lambda i: (0, i, 0)),
    out_shape=jax.ShapeDtypeStruct((2, NP, H // 2), jnp.float32),
)

_mlp = pl.pallas_call(
    _mlp_body,
    grid=(N // BM,),
    in_specs=[
        pl.BlockSpec((2, BM, H // 2), lambda i: (0, i, 0)),
        pl.BlockSpec((BM, 1), lambda i: (i, 0)),
        _full_spec((1, H)),
        _row_spec((BM, GENE)),
        _full_spec((GENE, H)),
        _full_spec((H, H)),
        _full_spec((1, H)),
        _full_spec((H, H2)),
        _full_spec((1, H2)),
    ],
    out_specs=_row_spec((BM, H2)),
    out_shape=jax.ShapeDtypeStruct((N, H2), jnp.float32),
)

_BP = 512

_pair_dot = pl.pallas_call(
    _dot_body,
    grid=(PPAD // _BP,),
    in_specs=[_row_spec((_BP, H2)), _row_spec((_BP, H2))],
    out_specs=pl.BlockSpec((_BP, 1), lambda i: (i, 0)),
    out_shape=jax.ShapeDtypeStruct((PPAD, 1), jnp.float32),
)


def _pad_reshape(a, total, fill, shape):
  pad = jnp.full((total - a.shape[0],), fill, dtype=a.dtype)
  return jnp.concatenate([a, pad]).reshape(shape)


def _pad_spread(a, total, shape):
  n = total - a.shape[0]
  pad = N + (jnp.arange(n, dtype=a.dtype) % (NP - N))
  return jnp.concatenate([a, pad]).reshape(shape)


@jax.jit
def kernel(x, adj, train_sample, llm_emb, W1, b1, W2, b2, M1, c1, M2, c2):
  src = adj[0]
  dst = adj[1]

  src_pad = _pad_spread(src, EPAD, (NT, EC, CH))
  src_idx = jnp.stack([src_pad, src_pad + NP])
  dst_idx = _pad_spread(dst, EPAD, (NT, EC, CH))
  deg_idx = jnp.stack([
      _pad_spread(dst[:E // 2], DPAD, (NT, DC, CH)),
      _pad_spread(dst[E // 2:], DPAD, (NT, DC, CH)),
  ])
  fidx = _pad_reshape(train_sample[:, 0], PPAD, 0, (NSC, NT, PC, CH))
  gidx = _pad_reshape(train_sample[:, 1], PPAD, 0, (NSC, NT, PC, CH))
  ones = jnp.ones((TROWS, CH), jnp.float32)

  degp = _deg_kernel(deg_idx, ones)

  xs1, dinv = _mm1(x, W1, degp)
  xs1 = xs1.reshape(NSC * NP, H // 2)
  acc1 = _edge_kernel(xs1, src_idx, dst_idx)

  xs2 = _mm2(acc1, W2, b1.reshape(1, H), dinv).reshape(NSC * NP, H // 2)
  acc2 = _edge_kernel(xs2, src_idx, dst_idx)

  z = _mlp(acc2, dinv, b2.reshape(1, H), llm_emb,
           M1[:, :GENE].T, M1[:, GENE:].T, c1.reshape(1, H),
           M2.T, c2.reshape(1, H2))

  tf, tg = _pair_kernel(z, fidx, gidx)
  pred = _pair_dot(tf, tg)
  return pred[:NTRAIN]


# speedup vs baseline: 1.8725x; 1.1797x over previous
import functools

import jax
import jax.numpy as jnp
from jax import lax
from jax.experimental import pallas as pl
from jax.experimental.pallas import tpu as pltpu
from jax.experimental.pallas import tpu_sc as plsc

N = 10000
E = 160000
D = 256
GENE = 512
H = 256
H2 = 128
NTRAIN = 20000

NSC = 2
NT = 16
CH = 128
TROWS = 632
NP = NT * TROWS
EC = 80
EPAD = NT * EC * CH
DC = 40
DPAD = NT * DC * CH
PC = 5
PPAD = NSC * NT * PC * CH

_mesh = plsc.VectorSubcoreMesh(core_axis_name="c", subcore_axis_name="s")


def _deg_body(idx_hbm, ones_hbm, out_hbm, idx_v, ones_v, accum):
  c = lax.axis_index("c")
  t = lax.axis_index("s")
  pltpu.sync_copy(idx_hbm.at[c, t], idx_v)
  pltpu.sync_copy(ones_hbm.at[pl.ds(0, CH)], ones_v)
  pltpu.sync_copy(ones_hbm, accum.at[pl.ds(t * TROWS, TROWS)])
  plsc.subcore_barrier()

  def body(j, carry):
    pltpu.sync_copy(ones_v, accum.at[idx_v.at[j]], add=True)
    return carry

  lax.fori_loop(0, DC, body, 0)
  plsc.subcore_barrier()
  pltpu.sync_copy(accum.at[pl.ds(t * TROWS, TROWS)],
                  out_hbm.at[c, pl.ds(t * TROWS, TROWS)])


_deg_kernel = pl.kernel(
    _deg_body,
    out_type=jax.ShapeDtypeStruct((NSC, NP, CH), jnp.float32),
    mesh=_mesh,
    scratch_types=[
        pltpu.VMEM((DC, CH), jnp.int32),
        pltpu.VMEM((CH, CH), jnp.float32),
        pltpu.VMEM_SHARED((NP, CH), jnp.float32),
    ],
)


G = 8
NG = EC // G


def _edge_body(xs_hbm, src_hbm, dst_hbm, out_hbm, sring, dring, buf0, buf1,
               accum, sem0, sem1):
  c = lax.axis_index("c")
  t = lax.axis_index("s")
  pltpu.sync_copy(xs_hbm.at[pl.ds(c * NP + t * TROWS, TROWS)],
                  accum.at[pl.ds(t * TROWS, TROWS)])
  plsc.subcore_barrier()

  bufs = (buf0, buf1)
  sems = (sem0, sem1)

  def group(g, carry):
    pltpu.sync_copy(src_hbm.at[c, t, pl.ds(g * G, G)], sring)
    pltpu.sync_copy(dst_hbm.at[t, pl.ds(g * G, G)], dring)
    cp = [None] * G
    cp[0] = pltpu.async_copy(xs_hbm.at[sring.at[0]], buf0, sem0)
    for k in range(G):
      if k + 1 < G:
        cp[k + 1] = pltpu.async_copy(xs_hbm.at[sring.at[k + 1]],
                                     bufs[(k + 1) % 2], sems[(k + 1) % 2])
      cp[k].wait()
      pltpu.sync_copy(bufs[k % 2], accum.at[dring.at[k]], add=True)
    return carry

  lax.fori_loop(0, NG, group, 0)
  plsc.subcore_barrier()
  pltpu.sync_copy(accum.at[pl.ds(t * TROWS, TROWS)],
                  out_hbm.at[c, pl.ds(t * TROWS, TROWS)])


_edge_kernel = pl.kernel(
    _edge_body,
    out_type=jax.ShapeDtypeStruct((NSC, NP, H // NSC), jnp.float32),
    mesh=_mesh,
    scratch_types=[
        pltpu.VMEM((G, CH), jnp.int32),
        pltpu.VMEM((G, CH), jnp.int32),
        pltpu.VMEM((CH, H // NSC), jnp.float32),
        pltpu.VMEM((CH, H // NSC), jnp.float32),
        pltpu.VMEM_SHARED((NP, H // NSC), jnp.float32),
        pltpu.SemaphoreType.DMA,
        pltpu.SemaphoreType.DMA,
    ],
)


def _pair_body(z_hbm, fidx_hbm, gidx_hbm, outf_hbm, outg_hbm, fidx_v, gidx_v,
               buf0, sem):
  c = lax.axis_index("c")
  t = lax.axis_index("s")
  w = c * NT + t
  pltpu.sync_copy(fidx_hbm.at[c, t], fidx_v)
  pltpu.sync_copy(gidx_hbm.at[c, t], gidx_v)

  def body(j, carry):
    base = w * (PC * CH) + j * CH
    pltpu.async_copy(z_hbm.at[fidx_v.at[j]], buf0, sem).wait()
    pltpu.sync_copy(buf0, outf_hbm.at[pl.ds(base, CH)])
    pltpu.async_copy(z_hbm.at[gidx_v.at[j]], buf0, sem).wait()
    pltpu.sync_copy(buf0, outg_hbm.at[pl.ds(base, CH)])
    return carry

  lax.fori_loop(0, PC, body, 0)


_pair_kernel = pl.kernel(
    _pair_body,
    out_type=[
        jax.ShapeDtypeStruct((PPAD, H2), jnp.float32),
        jax.ShapeDtypeStruct((PPAD, H2), jnp.float32),
    ],
    mesh=_mesh,
    scratch_types=[
        pltpu.VMEM((PC, CH), jnp.int32),
        pltpu.VMEM((PC, CH), jnp.int32),
        pltpu.VMEM((CH, H2), jnp.float32),
        pltpu.SemaphoreType.DMA,
    ],
)


BM = 400


def _mm1_body(x_ref, w_ref, degp_ref, out_ref, dinv_ref):
  deg = degp_ref[0, :, 0:1] + degp_ref[1, :, 0:1] - 1.0
  dinv = lax.rsqrt(deg)
  dinv_ref[...] = dinv
  xw = jnp.dot(x_ref[...], w_ref[...], preferred_element_type=jnp.float32)
  xs = xw * dinv
  out_ref[0] = xs[:, :H // 2]
  out_ref[1] = xs[:, H // 2:]


def _mm2_body(acc_ref, w_ref, b_ref, dinv_ref, out_ref):
  dinv = dinv_ref[...]
  h = jnp.concatenate([acc_ref[0], acc_ref[1]], axis=1) * dinv + b_ref[...]
  h = jnp.maximum(h, 0.0)
  xw = jnp.dot(h, w_ref[...], preferred_element_type=jnp.float32)
  xs = xw * dinv
  out_ref[0] = xs[:, :H // 2]
  out_ref[1] = xs[:, H // 2:]


def _lrelu(x):
  return jnp.where(x > 0, x, 0.01 * x)


def _mlp_body(acc_ref, dinv_ref, b2_ref, llm_ref, m1l_ref, m1r_ref, c1_ref,
              m2_ref, c2_ref, out_ref):
  dinv = dinv_ref[...]
  h = jnp.concatenate([acc_ref[0], acc_ref[1]], axis=1) * dinv + b2_ref[...]
  e = (jnp.dot(llm_ref[...], m1l_ref[...], preferred_element_type=jnp.float32)
       + jnp.dot(h, m1r_ref[...], preferred_element_type=jnp.float32)
       + c1_ref[...])
  z1 = _lrelu(e)
  z2 = _lrelu(jnp.dot(z1, m2_ref[...], preferred_element_type=jnp.float32)
              + c2_ref[...])
  out_ref[...] = z2


def _dot_body(tf_ref, tg_ref, out_ref):
  out_ref[...] = jnp.sum(tf_ref[...] * tg_ref[...], axis=1, keepdims=True)


def _row_spec(shape):
  nd = len(shape)
  return pl.BlockSpec((None,) * 0 + shape, lambda i: (i,) + (0,) * (nd - 1))


def _full_spec(shape):
  nd = len(shape)
  return pl.BlockSpec(shape, lambda i: (0,) * nd)


_mm1 = pl.pallas_call(
    _mm1_body,
    grid=(N // BM,),
    in_specs=[
        _row_spec((BM, D)),
        _full_spec((D, H)),
        pl.BlockSpec((2, BM, CH), lambda i: (0, i, 0)),
    ],
    out_specs=[
        pl.BlockSpec((2, BM, H // 2), lambda i: (0, i, 0)),
        pl.BlockSpec((BM, 1), lambda i: (i, 0)),
    ],
    out_shape=[
        jax.ShapeDtypeStruct((2, NP, H // 2), jnp.float32),
        jax.ShapeDtypeStruct((NP, 1), jnp.float32),
    ],
)

_mm2 = pl.pallas_call(
    _mm2_body,
    grid=(N // BM,),
    in_specs=[
        pl.BlockSpec((2, BM, H // 2), lambda i: (0, i, 0)),
        _full_spec((H, H)),
        _full_spec((1, H)),
        pl.BlockSpec((BM, 1), lambda i: (i, 0)),
    ],
    out_specs=pl.BlockSpec((2, BM, H // 2), lambda i: (0, i, 0)),
    out_shape=jax.ShapeDtypeStruct((2, NP, H // 2), jnp.float32),
)

_mlp = pl.pallas_call(
    _mlp_body,
    grid=(N // BM,),
    in_specs=[
        pl.BlockSpec((2, BM, H // 2), lambda i: (0, i, 0)),
        pl.BlockSpec((BM, 1), lambda i: (i, 0)),
        _full_spec((1, H)),
        _row_spec((BM, GENE)),
        _full_spec((GENE, H)),
        _full_spec((H, H)),
        _full_spec((1, H)),
        _full_spec((H, H2)),
        _full_spec((1, H2)),
    ],
    out_specs=_row_spec((BM, H2)),
    out_shape=jax.ShapeDtypeStruct((N, H2), jnp.float32),
)

_BP = 512

_pair_dot = pl.pallas_call(
    _dot_body,
    grid=(PPAD // _BP,),
    in_specs=[_row_spec((_BP, H2)), _row_spec((_BP, H2))],
    out_specs=pl.BlockSpec((_BP, 1), lambda i: (i, 0)),
    out_shape=jax.ShapeDtypeStruct((PPAD, 1), jnp.float32),
)


def _pad_reshape(a, total, fill, shape):
  pad = jnp.full((total - a.shape[0],), fill, dtype=a.dtype)
  return jnp.concatenate([a, pad]).reshape(shape)


def _pad_spread(a, total, shape):
  n = total - a.shape[0]
  pad = N + (jnp.arange(n, dtype=a.dtype) % (NP - N))
  return jnp.concatenate([a, pad]).reshape(shape)


@jax.jit
def kernel(x, adj, train_sample, llm_emb, W1, b1, W2, b2, M1, c1, M2, c2):
  src = adj[0]
  dst = adj[1]

  src_pad = _pad_spread(src, EPAD, (NT, EC, CH))
  src_idx = jnp.stack([src_pad, src_pad + NP])
  dst_idx = _pad_spread(dst, EPAD, (NT, EC, CH))
  deg_idx = jnp.stack([
      _pad_spread(dst[:E // 2], DPAD, (NT, DC, CH)),
      _pad_spread(dst[E // 2:], DPAD, (NT, DC, CH)),
  ])
  fidx = _pad_reshape(train_sample[:, 0], PPAD, 0, (NSC, NT, PC, CH))
  gidx = _pad_reshape(train_sample[:, 1], PPAD, 0, (NSC, NT, PC, CH))
  ones = jnp.ones((TROWS, CH), jnp.float32)

  degp = _deg_kernel(deg_idx, ones)

  xs1, dinv = _mm1(x, W1, degp)
  xs1 = xs1.reshape(NSC * NP, H // 2)
  acc1 = _edge_kernel(xs1, src_idx, dst_idx)

  xs2 = _mm2(acc1, W2, b1.reshape(1, H), dinv).reshape(NSC * NP, H // 2)
  acc2 = _edge_kernel(xs2, src_idx, dst_idx)

  z = _mlp(acc2, dinv, b2.reshape(1, H), llm_emb,
           M1[:, :GENE].T, M1[:, GENE:].T, c1.reshape(1, H),
           M2.T, c2.reshape(1, H2))

  tf, tg = _pair_kernel(z, fidx, gidx)
  pred = _pair_dot(tf, tg)
  return pred[:NTRAIN]


# speedup vs baseline: 2.1074x; 1.1254x over previous
import functools

import jax
import jax.numpy as jnp
from jax import lax
from jax.experimental import pallas as pl
from jax.experimental.pallas import tpu as pltpu
from jax.experimental.pallas import tpu_sc as plsc

N = 10000
E = 160000
D = 256
GENE = 512
H = 256
H2 = 128
NTRAIN = 20000

NSC = 2
NT = 16
CH = 128
TROWS = 632
NP = NT * TROWS
EC = 80
EPAD = NT * EC * CH
DC = 40
DPAD = NT * DC * CH
PC = 5
PPAD = NSC * NT * PC * CH

_mesh = plsc.VectorSubcoreMesh(core_axis_name="c", subcore_axis_name="s")


def _deg_body(idx_hbm, ones_hbm, out_hbm, idx_v, ones_v, accum):
  c = lax.axis_index("c")
  t = lax.axis_index("s")
  pltpu.sync_copy(idx_hbm.at[c, t], idx_v)
  pltpu.sync_copy(ones_hbm.at[pl.ds(0, CH)], ones_v)
  pltpu.sync_copy(ones_hbm, accum.at[pl.ds(t * TROWS, TROWS)])
  plsc.subcore_barrier()

  def body(j, carry):
    pltpu.sync_copy(ones_v, accum.at[idx_v.at[j]], add=True)
    return carry

  lax.fori_loop(0, DC, body, 0)
  plsc.subcore_barrier()
  pltpu.sync_copy(accum.at[pl.ds(t * TROWS, TROWS)],
                  out_hbm.at[c, pl.ds(t * TROWS, TROWS)])


_deg_kernel = pl.kernel(
    _deg_body,
    out_type=jax.ShapeDtypeStruct((NSC, NP, CH), jnp.float32),
    mesh=_mesh,
    scratch_types=[
        pltpu.VMEM((DC, CH), jnp.int32),
        pltpu.VMEM((CH, CH), jnp.float32),
        pltpu.VMEM_SHARED((NP, CH), jnp.float32),
    ],
)


G = 8
NG = EC // G


def _edge_body(xs_hbm, src_hbm, dst_hbm, out_hbm, sring, dring, buf0, buf1,
               accum, sem0, sem1):
  c = lax.axis_index("c")
  t = lax.axis_index("s")
  pltpu.sync_copy(xs_hbm.at[pl.ds(c * NP + t * TROWS, TROWS)],
                  accum.at[pl.ds(t * TROWS, TROWS)])
  plsc.subcore_barrier()

  bufs = (buf0, buf1)
  sems = (sem0, sem1)

  def group(g, carry):
    pltpu.sync_copy(src_hbm.at[c, t, pl.ds(g * G, G)], sring)
    pltpu.sync_copy(dst_hbm.at[t, pl.ds(g * G, G)], dring)
    cp = [None] * G
    cp[0] = pltpu.async_copy(xs_hbm.at[sring.at[0]], buf0, sem0)
    for k in range(G):
      if k + 1 < G:
        cp[k + 1] = pltpu.async_copy(xs_hbm.at[sring.at[k + 1]],
                                     bufs[(k + 1) % 2], sems[(k + 1) % 2])
      cp[k].wait()
      pltpu.sync_copy(bufs[k % 2], accum.at[dring.at[k]], add=True)
    return carry

  lax.fori_loop(0, NG, group, 0)
  plsc.subcore_barrier()
  pltpu.sync_copy(accum.at[pl.ds(t * TROWS, TROWS)],
                  out_hbm.at[c, pl.ds(t * TROWS, TROWS)])


_edge_kernel = pl.kernel(
    _edge_body,
    out_type=jax.ShapeDtypeStruct((NSC, NP, H // NSC), jnp.float32),
    mesh=_mesh,
    scratch_types=[
        pltpu.VMEM((G, CH), jnp.int32),
        pltpu.VMEM((G, CH), jnp.int32),
        pltpu.VMEM((CH, H // NSC), jnp.float32),
        pltpu.VMEM((CH, H // NSC), jnp.float32),
        pltpu.VMEM_SHARED((NP, H // NSC), jnp.float32),
        pltpu.SemaphoreType.DMA,
        pltpu.SemaphoreType.DMA,
    ],
)


def _pair_body(z_hbm, fidx_hbm, gidx_hbm, outf_hbm, outg_hbm, fidx_v, gidx_v,
               buf0, buf1, sem0, sem1):
  c = lax.axis_index("c")
  t = lax.axis_index("s")
  w = c * NT + t
  pltpu.sync_copy(fidx_hbm.at[c, t], fidx_v)
  pltpu.sync_copy(gidx_hbm.at[c, t], gidx_v)

  idxs = [(fidx_v, 0)] * PC
  bufs = (buf0, buf1)
  sems = (sem0, sem1)
  work = []
  for j in range(PC):
    work.append((fidx_v, j, 0))
    work.append((gidx_v, j, 1))

  def start(k):
    iv, j, _ = work[k]
    return pltpu.async_copy(z_hbm.at[iv.at[j]], bufs[k % 2], sems[k % 2])

  cp = [None] * len(work)
  cp[0] = start(0)
  for k in range(len(work)):
    if k + 1 < len(work):
      cp[k + 1] = start(k + 1)
    cp[k].wait()
    _, j, which = work[k]
    base = w * (PC * CH) + j * CH
    if which == 0:
      pltpu.sync_copy(bufs[k % 2], outf_hbm.at[pl.ds(base, CH)])
    else:
      pltpu.sync_copy(bufs[k % 2], outg_hbm.at[pl.ds(base, CH)])


_pair_kernel = pl.kernel(
    _pair_body,
    out_type=[
        jax.ShapeDtypeStruct((PPAD, H2), jnp.float32),
        jax.ShapeDtypeStruct((PPAD, H2), jnp.float32),
    ],
    mesh=_mesh,
    scratch_types=[
        pltpu.VMEM((PC, CH), jnp.int32),
        pltpu.VMEM((PC, CH), jnp.int32),
        pltpu.VMEM((CH, H2), jnp.float32),
        pltpu.VMEM((CH, H2), jnp.float32),
        pltpu.SemaphoreType.DMA,
        pltpu.SemaphoreType.DMA,
    ],
)


BM = 400


def _mm1_body(x_ref, w_ref, degp_ref, out_ref, dinv_ref):
  deg = degp_ref[0, :, 0:1] + degp_ref[1, :, 0:1] - 1.0
  dinv = lax.rsqrt(deg)
  dinv_ref[...] = dinv
  xw = jnp.dot(x_ref[...], w_ref[...], preferred_element_type=jnp.float32)
  xs = xw * dinv
  out_ref[0] = xs[:, :H // 2]
  out_ref[1] = xs[:, H // 2:]


def _mm2_body(acc_ref, w_ref, b_ref, dinv_ref, out_ref):
  dinv = dinv_ref[...]
  h = jnp.concatenate([acc_ref[0], acc_ref[1]], axis=1) * dinv + b_ref[...]
  h = jnp.maximum(h, 0.0)
  xw = jnp.dot(h, w_ref[...], preferred_element_type=jnp.float32)
  xs = xw * dinv
  out_ref[0] = xs[:, :H // 2]
  out_ref[1] = xs[:, H // 2:]


def _lrelu(x):
  return jnp.where(x > 0, x, 0.01 * x)


def _mlp_body(acc_ref, dinv_ref, b2_ref, llm_ref, m1l_ref, m1r_ref, c1_ref,
              m2_ref, c2_ref, out_ref):
  dinv = dinv_ref[...]
  h = jnp.concatenate([acc_ref[0], acc_ref[1]], axis=1) * dinv + b2_ref[...]
  e = (jnp.dot(llm_ref[...], m1l_ref[...], preferred_element_type=jnp.float32)
       + jnp.dot(h, m1r_ref[...], preferred_element_type=jnp.float32)
       + c1_ref[...])
  z1 = _lrelu(e)
  z2 = _lrelu(jnp.dot(z1, m2_ref[...], preferred_element_type=jnp.float32)
              + c2_ref[...])
  out_ref[...] = z2


def _dot_body(tf_ref, tg_ref, out_ref):
  out_ref[...] = jnp.sum(tf_ref[...] * tg_ref[...], axis=1, keepdims=True)


def _row_spec(shape):
  nd = len(shape)
  return pl.BlockSpec((None,) * 0 + shape, lambda i: (i,) + (0,) * (nd - 1))


def _full_spec(shape):
  nd = len(shape)
  return pl.BlockSpec(shape, lambda i: (0,) * nd)


_mm1 = pl.pallas_call(
    _mm1_body,
    grid=(N // BM,),
    in_specs=[
        _row_spec((BM, D)),
        _full_spec((D, H)),
        pl.BlockSpec((2, BM, CH), lambda i: (0, i, 0)),
    ],
    out_specs=[
        pl.BlockSpec((2, BM, H // 2), lambda i: (0, i, 0)),
        pl.BlockSpec((BM, 1), lambda i: (i, 0)),
    ],
    out_shape=[
        jax.ShapeDtypeStruct((2, NP, H // 2), jnp.float32),
        jax.ShapeDtypeStruct((NP, 1), jnp.float32),
    ],
)

_mm2 = pl.pallas_call(
    _mm2_body,
    grid=(N // BM,),
    in_specs=[
        pl.BlockSpec((2, BM, H // 2), lambda i: (0, i, 0)),
        _full_spec((H, H)),
        _full_spec((1, H)),
        pl.BlockSpec((BM, 1), lambda i: (i, 0)),
    ],
    out_specs=pl.BlockSpec((2, BM, H // 2), lambda i: (0, i, 0)),
    out_shape=jax.ShapeDtypeStruct((2, NP, H // 2), jnp.float32),
)

_mlp = pl.pallas_call(
    _mlp_body,
    grid=(N // BM,),
    in_specs=[
        pl.BlockSpec((2, BM, H // 2), lambda i: (0, i, 0)),
        pl.BlockSpec((BM, 1), lambda i: (i, 0)),
        _full_spec((1, H)),
        _row_spec((BM, GENE)),
        _full_spec((GENE, H)),
        _full_spec((H, H)),
        _full_spec((1, H)),
        _full_spec((H, H2)),
        _full_spec((1, H2)),
    ],
    out_specs=_row_spec((BM, H2)),
    out_shape=jax.ShapeDtypeStruct((N, H2), jnp.float32),
)

_BP = 512

_pair_dot = pl.pallas_call(
    _dot_body,
    grid=(PPAD // _BP,),
    in_specs=[_row_spec((_BP, H2)), _row_spec((_BP, H2))],
    out_specs=pl.BlockSpec((_BP, 1), lambda i: (i, 0)),
    out_shape=jax.ShapeDtypeStruct((PPAD, 1), jnp.float32),
)


def _pad_reshape(a, total, fill, shape):
  pad = jnp.full((total - a.shape[0],), fill, dtype=a.dtype)
  return jnp.concatenate([a, pad]).reshape(shape)


def _pad_spread(a, total, shape):
  n = total - a.shape[0]
  pad = N + (jnp.arange(n, dtype=a.dtype) % (NP - N))
  return jnp.concatenate([a, pad]).reshape(shape)


@jax.jit
def kernel(x, adj, train_sample, llm_emb, W1, b1, W2, b2, M1, c1, M2, c2):
  src = adj[0]
  dst = adj[1]

  src_pad = _pad_spread(src, EPAD, (NT, EC, CH))
  src_idx = jnp.stack([src_pad, src_pad + NP])
  dst_idx = _pad_spread(dst, EPAD, (NT, EC, CH))
  deg_idx = jnp.stack([
      _pad_spread(dst[:E // 2], DPAD, (NT, DC, CH)),
      _pad_spread(dst[E // 2:], DPAD, (NT, DC, CH)),
  ])
  nppad = PPAD - NTRAIN
  pspread = (jnp.arange(nppad, dtype=jnp.int32) * 61) % N
  fidx = jnp.concatenate([train_sample[:, 0], pspread]).reshape(
      (NSC, NT, PC, CH))
  gidx = jnp.concatenate([train_sample[:, 1], pspread]).reshape(
      (NSC, NT, PC, CH))
  ones = jnp.ones((TROWS, CH), jnp.float32)

  degp = _deg_kernel(deg_idx, ones)

  xs1, dinv = _mm1(x, W1, degp)
  xs1 = xs1.reshape(NSC * NP, H // 2)
  acc1 = _edge_kernel(xs1, src_idx, dst_idx)

  xs2 = _mm2(acc1, W2, b1.reshape(1, H), dinv).reshape(NSC * NP, H // 2)
  acc2 = _edge_kernel(xs2, src_idx, dst_idx)

  z = _mlp(acc2, dinv, b2.reshape(1, H), llm_emb,
           M1[:, :GENE].T, M1[:, GENE:].T, c1.reshape(1, H),
           M2.T, c2.reshape(1, H2))

  tf, tg = _pair_kernel(z, fidx, gidx)
  pred = _pair_dot(tf, tg)
  return pred[:NTRAIN]
